# Initial kernel scaffold; baseline (speedup 1.0000x reference)
import jax
import jax.numpy as jnp
from jax import lax
from jax.experimental import pallas as pl
from jax.experimental.pallas import tpu as pltpu
from jax.experimental.pallas import tpu_sc as plsc

PRE_SEQ_LEN = 128
HIDDEN = 1024
NUM_LAYERS = 24
OUT_DIM = NUM_LAYERS * 2 * HIDDEN
BATCH = 4

SPLIT = 8
DC = OUT_DIM // SPLIT
NB = BATCH * PRE_SEQ_LEN
NROWS = NB * SPLIT

NC, NS, L = 2, 16, 16
NW = NC * NS
B_PER_W = NB // NW
R_PER_W = B_PER_W * SPLIT
G = 8
N_BATCH = R_PER_W // G


def _body(table, idx_hbm, out, idx_v, newidx, buf, sem, wsem):
    wid = lax.axis_index("s") * NC + lax.axis_index("c")
    base = wid * B_PER_W

    pltpu.sync_copy(idx_hbm.at[pl.ds(base, B_PER_W)], idx_v)

    for jb in range(R_PER_W // L):
        j = jb * L + lax.iota(jnp.int32, L)
        b = lax.shift_right_logical(j, 3)
        c = lax.bitwise_and(j, 7)
        g = plsc.load_gather(idx_v, [b])
        newidx[pl.ds(jb * L, L)] = g * SPLIT + c

    for gb in range(N_BATCH):
        pltpu.async_copy(table.at[newidx.at[pl.ds(gb * G, G)]], buf, sem).wait()
        pltpu.async_copy(buf, out.at[pl.ds(wid * R_PER_W + gb * G, G)], wsem).wait()


@jax.jit
def _sc_gather(table_r, idx):
    mesh = plsc.VectorSubcoreMesh(core_axis_name="c", subcore_axis_name="s")
    k = pl.kernel(
        _body,
        out_type=jax.ShapeDtypeStruct((NROWS, DC), jnp.float32),
        mesh=mesh,
        scratch_types=[
            pltpu.VMEM((B_PER_W,), jnp.int32),
            pltpu.VMEM((R_PER_W,), jnp.int32),
            pltpu.VMEM((G, DC), jnp.float32),
            pltpu.SemaphoreType.DMA,
            pltpu.SemaphoreType.DMA,
        ],
    )
    return k(table_r, idx)


def kernel(prefix, embedding_weight):
    idx = prefix.reshape(NB)
    table_r = embedding_weight.reshape(PRE_SEQ_LEN * SPLIT, DC)
    out = _sc_gather(table_r, idx)
    return out.reshape(BATCH, PRE_SEQ_LEN, OUT_DIM)


# speedup vs baseline: 1.5309x; 1.5309x over previous
import jax
import jax.numpy as jnp
from jax import lax
from jax.experimental import pallas as pl
from jax.experimental.pallas import tpu as pltpu
from jax.experimental.pallas import tpu_sc as plsc

PRE_SEQ_LEN = 128
HIDDEN = 1024
NUM_LAYERS = 24
OUT_DIM = NUM_LAYERS * 2 * HIDDEN
BATCH = 4

NB = BATCH * PRE_SEQ_LEN
SPLIT = 8
DC = OUT_DIM // SPLIT

NC, NS, L = 2, 16, 16
NW = NC * NS
B_PER_W = NB // NW


def _body(table, idx_hbm, out, idx_v, buf, sem, wsem):
    wid = lax.axis_index("s") * NC + lax.axis_index("c")
    base = wid * B_PER_W

    pltpu.sync_copy(idx_hbm.at[pl.ds(base, B_PER_W)], idx_v)

    for c in range(SPLIT):
        pltpu.async_copy(
            table.at[idx_v, pl.ds(c * DC, DC)], buf, sem
        ).wait()
        pltpu.async_copy(
            buf, out.at[pl.ds(base, B_PER_W), pl.ds(c * DC, DC)], wsem
        ).wait()


@jax.jit
def _sc_gather(table, idx):
    mesh = plsc.VectorSubcoreMesh(core_axis_name="c", subcore_axis_name="s")
    k = pl.kernel(
        _body,
        out_type=jax.ShapeDtypeStruct((NB, OUT_DIM), jnp.float32),
        mesh=mesh,
        scratch_types=[
            pltpu.VMEM((B_PER_W,), jnp.int32),
            pltpu.VMEM((B_PER_W, DC), jnp.float32),
            pltpu.SemaphoreType.DMA,
            pltpu.SemaphoreType.DMA,
        ],
    )
    return k(table, idx)


def kernel(prefix, embedding_weight):
    idx = prefix.reshape(NB)
    out = _sc_gather(embedding_weight, idx)
    return out.reshape(BATCH, PRE_SEQ_LEN, OUT_DIM)


# speedup vs baseline: 1.6328x; 1.0666x over previous
import jax
import jax.numpy as jnp
from jax import lax
from jax.experimental import pallas as pl
from jax.experimental.pallas import tpu as pltpu
from jax.experimental.pallas import tpu_sc as plsc

PRE_SEQ_LEN = 128
HIDDEN = 1024
NUM_LAYERS = 24
OUT_DIM = NUM_LAYERS * 2 * HIDDEN
BATCH = 4

NB = BATCH * PRE_SEQ_LEN
SPLIT = 16
DC = OUT_DIM // SPLIT

NC, NS = 2, 16
NW = NC * NS
B_PER_W = NB // NW


def _body(table, idx_hbm, out, idx_v, buf0, buf1, gsem0, gsem1, wsem0, wsem1):
    wid = lax.axis_index("s") * NC + lax.axis_index("c")
    base = wid * B_PER_W

    bufs = (buf0, buf1)
    gsems = (gsem0, gsem1)
    wsems = (wsem0, wsem1)

    pltpu.sync_copy(idx_hbm.at[pl.ds(base, B_PER_W)], idx_v)

    gd = [None, None]
    wd = [None, None]
    for c in range(SPLIT + 1):
        if c < SPLIT:
            b = c % 2
            if wd[b] is not None:
                wd[b].wait()
            gd[b] = pltpu.async_copy(
                table.at[idx_v, pl.ds(c * DC, DC)], bufs[b], gsems[b]
            )
        if c >= 1:
            b = (c - 1) % 2
            gd[b].wait()
            wd[b] = pltpu.async_copy(
                bufs[b],
                out.at[pl.ds(base, B_PER_W), pl.ds((c - 1) * DC, DC)],
                wsems[b],
            )
    wd[0].wait()
    wd[1].wait()


@jax.jit
def _sc_gather(table, idx):
    mesh = plsc.VectorSubcoreMesh(core_axis_name="c", subcore_axis_name="s")
    k = pl.kernel(
        _body,
        out_type=jax.ShapeDtypeStruct((NB, OUT_DIM), jnp.float32),
        mesh=mesh,
        scratch_types=[
            pltpu.VMEM((B_PER_W,), jnp.int32),
            pltpu.VMEM((B_PER_W, DC), jnp.float32),
            pltpu.VMEM((B_PER_W, DC), jnp.float32),
            pltpu.SemaphoreType.DMA,
            pltpu.SemaphoreType.DMA,
            pltpu.SemaphoreType.DMA,
            pltpu.SemaphoreType.DMA,
        ],
    )
    return k(table, idx)


def kernel(prefix, embedding_weight):
    idx = prefix.reshape(NB)
    out = _sc_gather(embedding_weight, idx)
    return out.reshape(BATCH, PRE_SEQ_LEN, OUT_DIM)
